# baseline (speedup 1.0000x reference)
import math

import jax
import jax.numpy as jnp
from jax.experimental import pallas as pl
from jax.experimental.pallas import tpu as pltpu

_MSL = 4
_NEG = -jnp.inf


def _shift_down(x, d):
    if d == 0:
        return x
    n = x.shape[0]
    return jnp.concatenate([x[n - d:], x[:n - d]], axis=0)


def _body(h_ref, th_ref, tb_ref, xb_ref, le_ref, w1_ref, b1_ref,
          w2_ref, b2_ref, out_ref):
    PB, S, H = h_ref.shape
    T = th_ref.shape[1]
    f32 = jnp.float32

    Lc = jnp.dot(le_ref[...], w1_ref[3 * H:, :],
                 preferred_element_type=f32)
    ridx = jax.lax.broadcasted_iota(jnp.int32, (S, 1), 0)
    Lcb = Lc + b1_ref[...]
    b2 = b2_ref[...]

    CH = 8
    R = S // CH
    W = 8

    for pb in range(PB):
        h = h_ref[pb]
        th = th_ref[pb]
        tnrm = jnp.maximum(jnp.sqrt(jnp.sum(th * th, axis=1, keepdims=True)),
                           1e-12)
        tn = th / tnrm
        tbias = tb_ref[pb]

        chunk_cols = []
        for c in range(CH):
            r0 = c * R
            if c < CH - 1:
                hw = h[r0:r0 + R + W]
            else:
                hw = jnp.concatenate([h[r0:r0 + R], h[0:W]], axis=0)
            Ac = jnp.dot(hw[0:R], w1_ref[0:H, :], preferred_element_type=f32)
            Ew = jnp.dot(hw, w1_ref[H:2 * H, :], preferred_element_type=f32)
            Mw = jnp.dot(hw, w1_ref[2 * H:3 * H, :], preferred_element_type=f32)
            cols = []
            Msc = None
            for l in range(1, _MSL + 1):
                d = l - 1
                if d == 0:
                    Msc = Mw[0:R]
                else:
                    Msc = Msc + Mw[d:R + d]
                pre = (Ac + Ew[d:R + d] + Msc * (1.0 / l)
                       + Lcb[l:l + 1, :])
                h1 = 0.5 * pre * (1.0 +
                                  jax.lax.erf(pre * (1.0 / math.sqrt(2.0))))
                sv = jnp.dot(h1, w2_ref[...], preferred_element_type=f32) + b2
                nrm = jnp.maximum(
                    jnp.sqrt(jnp.sum(sv * sv, axis=1, keepdims=True)), 1e-12)
                P = jax.lax.dot_general(sv, tn, (((1,), (1,)), ((), ())),
                                        preferred_element_type=f32)
                P = P + tbias
                sc = jnp.max(P, axis=1, keepdims=True) / nrm
                rloc = ridx[r0:r0 + R]
                sc = jnp.where(rloc <= S - l, sc, _NEG)
                cols.append(sc)
            chunk_cols.append(jnp.concatenate(cols, axis=1))

        scm = jnp.concatenate(chunk_cols, axis=0)
        tok = jnp.full((S, 1), _NEG, dtype=f32)
        for d in range(_MSL):
            part = jnp.max(_shift_down(scm[:, d:], d), axis=1, keepdims=True)
            tok = jnp.maximum(tok, jnp.where(ridx >= d, part, _NEG))

        out_ref[pb, 0, :] = jnp.reshape(tok, (S,)) + xb_ref[pb, 0, :]


def kernel(hidden_states, topic_hidden, topic_mask, text_mask,
           length_embedding, W1, b1, W2, b2):
    B, S, H = hidden_states.shape
    T = topic_hidden.shape[1]
    f32 = jnp.float32

    tb = jnp.where(topic_mask, 0.0, _NEG).astype(f32).reshape(B, 1, T)
    xb = jnp.where(text_mask, 0.0, _NEG).astype(f32).reshape(B, 1, S)
    b1r = b1.reshape(1, H).astype(f32)
    b2r = b2.reshape(1, H).astype(f32)

    PB = 2
    call = pl.pallas_call(
        _body,
        grid=(B // PB,),
        in_specs=[
            pl.BlockSpec((PB, S, H), lambda g: (g, 0, 0)),
            pl.BlockSpec((PB, T, H), lambda g: (g, 0, 0)),
            pl.BlockSpec((PB, 1, T), lambda g: (g, 0, 0)),
            pl.BlockSpec((PB, 1, S), lambda g: (g, 0, 0)),
            pl.BlockSpec(length_embedding.shape, lambda g: (0, 0)),
            pl.BlockSpec(W1.shape, lambda g: (0, 0)),
            pl.BlockSpec((1, H), lambda g: (0, 0)),
            pl.BlockSpec(W2.shape, lambda g: (0, 0)),
            pl.BlockSpec((1, H), lambda g: (0, 0)),
        ],
        out_specs=pl.BlockSpec((PB, 1, S), lambda g: (g, 0, 0)),
        out_shape=jax.ShapeDtypeStruct((B, 1, S), f32),
        compiler_params=pltpu.CompilerParams(
            dimension_semantics=("parallel",),
            vmem_limit_bytes=110 * 1024 * 1024,
        ),
    )
    out = call(hidden_states.astype(f32), topic_hidden.astype(f32), tb, xb,
               length_embedding.astype(f32), W1.astype(f32), b1r,
               W2.astype(f32), b2r)
    return out.reshape(B, S)


# speedup vs baseline: 1.3163x; 1.3163x over previous
import math

import jax
import jax.numpy as jnp
from jax.experimental import pallas as pl
from jax.experimental.pallas import tpu as pltpu

_MSL = 4
_NEG = -jnp.inf


def _shift_down(x, d):
    if d == 0:
        return x
    n = x.shape[0]
    return jnp.concatenate([x[n - d:], x[:n - d]], axis=0)


def _body(h_ref, th_ref, tb_ref, xb_ref, le_ref, w1_ref, b1_ref,
          w2_ref, b2_ref, out_ref):
    PB, S, H = h_ref.shape
    T = th_ref.shape[1]
    f32 = jnp.float32

    Lc = jnp.dot(le_ref[...], w1_ref[3 * H:, :],
                 preferred_element_type=f32)
    ridx = jax.lax.broadcasted_iota(jnp.int32, (S, 1), 0)
    Lcb = Lc + b1_ref[...]
    b2 = b2_ref[...]

    CH = 8
    R = S // CH
    W = 8

    for pb in range(PB):
        h = h_ref[pb]
        th = th_ref[pb]
        tnrm = jnp.maximum(jnp.sqrt(jnp.sum(th * th, axis=1, keepdims=True)),
                           1e-12)
        tn = th / tnrm
        tbias = tb_ref[pb]

        chunk_cols = []
        for c in range(CH):
            r0 = c * R
            if c < CH - 1:
                hw = h[r0:r0 + R + W]
            else:
                hw = jnp.concatenate([h[r0:r0 + R], h[0:W]], axis=0)
            Ac = jnp.dot(hw[0:R], w1_ref[0:H, :], preferred_element_type=f32)
            Ew = jnp.dot(hw, w1_ref[H:2 * H, :], preferred_element_type=f32)
            Mw = jnp.dot(hw, w1_ref[2 * H:3 * H, :], preferred_element_type=f32)
            cols = []
            Msc = None
            for l in range(1, _MSL + 1):
                d = l - 1
                if d == 0:
                    Msc = Mw[0:R]
                else:
                    Msc = Msc + Mw[d:R + d]
                pre = (Ac + Ew[d:R + d] + Msc * (1.0 / l)
                       + Lcb[l:l + 1, :])
                h1 = 0.5 * pre * (1.0 +
                                  jax.lax.erf(pre * (1.0 / math.sqrt(2.0))))
                sv = jnp.dot(h1, w2_ref[...], preferred_element_type=f32) + b2
                nrm = jnp.maximum(
                    jnp.sqrt(jnp.sum(sv * sv, axis=1, keepdims=True)), 1e-12)
                P = jax.lax.dot_general(sv, tn, (((1,), (1,)), ((), ())),
                                        preferred_element_type=f32)
                P = P + tbias
                sc = jnp.max(P, axis=1, keepdims=True) / nrm
                rloc = ridx[r0:r0 + R]
                sc = jnp.where(rloc <= S - l, sc, _NEG)
                cols.append(sc)
            chunk_cols.append(jnp.concatenate(cols, axis=1))

        scm = jnp.concatenate(chunk_cols, axis=0)
        tok = jnp.full((S, 1), _NEG, dtype=f32)
        for d in range(_MSL):
            part = jnp.max(_shift_down(scm[:, d:], d), axis=1, keepdims=True)
            tok = jnp.maximum(tok, jnp.where(ridx >= d, part, _NEG))

        out_ref[pb, 0, :] = jnp.reshape(tok, (S,)) + xb_ref[pb, 0, :]


def kernel(hidden_states, topic_hidden, topic_mask, text_mask,
           length_embedding, W1, b1, W2, b2):
    B, S, H = hidden_states.shape
    T = topic_hidden.shape[1]
    f32 = jnp.float32

    tb = jnp.where(topic_mask, 0.0, _NEG).astype(f32).reshape(B, 1, T)
    xb = jnp.where(text_mask, 0.0, _NEG).astype(f32).reshape(B, 1, S)
    b1r = b1.reshape(1, H).astype(f32)
    b2r = b2.reshape(1, H).astype(f32)

    PB = 1
    call = pl.pallas_call(
        _body,
        grid=(B // PB,),
        in_specs=[
            pl.BlockSpec((PB, S, H), lambda g: (g, 0, 0)),
            pl.BlockSpec((PB, T, H), lambda g: (g, 0, 0)),
            pl.BlockSpec((PB, 1, T), lambda g: (g, 0, 0)),
            pl.BlockSpec((PB, 1, S), lambda g: (g, 0, 0)),
            pl.BlockSpec(length_embedding.shape, lambda g: (0, 0)),
            pl.BlockSpec(W1.shape, lambda g: (0, 0)),
            pl.BlockSpec((1, H), lambda g: (0, 0)),
            pl.BlockSpec(W2.shape, lambda g: (0, 0)),
            pl.BlockSpec((1, H), lambda g: (0, 0)),
        ],
        out_specs=pl.BlockSpec((PB, 1, S), lambda g: (g, 0, 0)),
        out_shape=jax.ShapeDtypeStruct((B, 1, S), f32),
        compiler_params=pltpu.CompilerParams(
            dimension_semantics=("parallel",),
            vmem_limit_bytes=110 * 1024 * 1024,
        ),
    )
    out = call(hidden_states.astype(f32), topic_hidden.astype(f32), tb, xb,
               length_embedding.astype(f32), W1.astype(f32), b1r,
               W2.astype(f32), b2r)
    return out.reshape(B, S)
